# baseline (speedup 1.0000x reference)
import functools

import jax
import jax.numpy as jnp
from jax import lax
from jax.experimental import pallas as pl
from jax.experimental.pallas import tpu as pltpu
from jax.experimental.pallas import tpu_sc as plsc

EMB = 32
NT = 4
HIST = 200
BB = 128
CH = 2
LANES = 16


def _make_lookup(n_batch):
    info = plsc.get_sparse_core_info()
    nw = info.num_cores * info.num_subcores
    assert BB * nw == n_batch
    n_chunks = HIST // CH
    assert n_chunks * CH == HIST and n_chunks % 2 == 0

    mesh = plsc.VectorSubcoreMesh(core_axis_name="c", subcore_axis_name="s")

    @functools.partial(
        pl.kernel,
        out_type=jax.ShapeDtypeStruct((HIST, EMB, n_batch), jnp.float32),
        mesh=mesh,
        scratch_types=[
            pltpu.VMEM((2, CH, NT, BB), jnp.int32),
            pltpu.VMEM((2, CH, NT, BB, EMB), jnp.float32),
            pltpu.VMEM((2, CH, EMB, BB), jnp.float32),
            pltpu.VMEM((LANES * EMB,), jnp.float32),
            pltpu.VMEM((LANES * LANES,), jnp.float32),
            pltpu.SemaphoreType.DMA,
            pltpu.SemaphoreType.DMA,
            pltpu.SemaphoreType.DMA,
            pltpu.SemaphoreType.DMA,
            pltpu.SemaphoreType.DMA,
            pltpu.SemaphoreType.DMA,
        ],
        compiler_params=pltpu.CompilerParams(
            use_tc_tiling_on_sc=False, needs_layout_passes=False),
    )
    def lookup(idx_hbm, w0, w1, w2, w3, out_hbm,
               idx_v, rows_v, out_v, tsum_v, ttr_v,
               gs0, gs1, is0, is1, os0, os1):
        tables = (w0, w1, w2, w3)
        gsem = (gs0, gs1)
        isem = (is0, is1)
        osem = (os0, os1)
        wid = lax.axis_index("s") * info.num_cores + lax.axis_index("c")
        b0 = wid * BB
        lanes = lax.iota(jnp.int32, LANES)
        diag0 = [lanes * EMB + lax.rem(lanes + d, LANES)
                 for d in range(LANES)]
        rotflat = [lax.rem(lanes * (LANES - 1) + e, LANES) * LANES + lanes
                   for e in range(LANES)]

        def idx_copies(k, sp):
            return [pltpu.make_async_copy(
                idx_hbm.at[k * CH + c, t, pl.ds(b0, BB)],
                idx_v.at[sp, c, t], isem[sp])
                for c in range(CH) for t in range(NT)]

        def gather_copies(k, sp):
            del k
            return [pltpu.make_async_copy(
                tables[t].at[idx_v.at[sp, c, t]],
                rows_v.at[sp, c, t], gsem[sp])
                for c in range(CH) for t in range(NT)]

        def out_copies(k, sp):
            return [pltpu.make_async_copy(
                out_v.at[sp, c],
                out_hbm.at[k * CH + c, :, pl.ds(b0, BB)], osem[sp])
                for c in range(CH)]

        for c in idx_copies(0, 0):
            c.start()
        for c in idx_copies(0, 0):
            c.wait()
        for c in gather_copies(0, 0):
            c.start()
        for c in idx_copies(1, 1):
            c.start()

        def pair_body(kk, carry):
            for s in (0, 1):
                k = 2 * kk + s
                sn = 1 - s
                for c in gather_copies(k, s):
                    c.wait()

                @pl.when(k + 2 < n_chunks)
                def _prefetch_idx():
                    for c in idx_copies(k + 2, s):
                        c.start()

                @pl.when(k + 1 < n_chunks)
                def _fire_next():
                    for c in idx_copies(k + 1, sn):
                        c.wait()
                    for c in gather_copies(k + 1, sn):
                        c.start()

                @pl.when(k >= 2)
                def _reclaim_out():
                    for c in out_copies(k - 2, s):
                        c.wait()

                for c in range(CH):
                    def bb_body(bb, carry2, c=c):
                        for j in range(LANES):
                            jb = bb * LANES + j
                            for h in (0, LANES):
                                d = pl.ds(h, LANES)
                                s01 = (rows_v[s, c, 0, jb, d]
                                       + rows_v[s, c, 1, jb, d])
                                s23 = (rows_v[s, c, 2, jb, d]
                                       + rows_v[s, c, 3, jb, d])
                                tsum_v[pl.ds(j * EMB + h, LANES)] = (
                                    (s01 + s23) * jnp.float32(0.25))
                        for eb in range(EMB // LANES):
                            for dd in range(LANES):
                                v = plsc.load_gather(
                                    tsum_v, [diag0[dd] + eb * LANES])
                                ttr_v[pl.ds(dd * LANES, LANES)] = v
                            for e in range(LANES):
                                v = plsc.load_gather(ttr_v, [rotflat[e]])
                                out_v[s, c, eb * LANES + e,
                                      pl.ds(bb * LANES, LANES)] = v
                        return carry2

                    lax.fori_loop(0, BB // LANES, bb_body, 0)
                for c in out_copies(k, s):
                    c.start()
            return carry

        lax.fori_loop(0, n_chunks // 2, pair_body, 0)
        for c in out_copies(n_chunks - 2, 0):
            c.wait()
        for c in out_copies(n_chunks - 1, 1):
            c.wait()

    return lookup


def kernel(poi_path, W0, W1, W2, W3):
    b, h, nt = poi_path.shape
    idx_t = jnp.transpose(poi_path, (1, 2, 0))
    out = _make_lookup(b)(idx_t, W0, W1, W2, W3)
    return jnp.transpose(out, (2, 0, 1))


# speedup vs baseline: 1.3647x; 1.3647x over previous
import functools

import jax
import jax.numpy as jnp
from jax import lax
from jax.experimental import pallas as pl
from jax.experimental.pallas import tpu as pltpu
from jax.experimental.pallas import tpu_sc as plsc

EMB = 32
NT = 4
HIST = 200
BB = 128
CH = 2
LANES = 16


def _make_lookup(n_batch):
    info = plsc.get_sparse_core_info()
    nw = info.num_cores * info.num_subcores
    assert BB * nw == n_batch
    n_chunks = HIST // CH
    assert n_chunks * CH == HIST and n_chunks % 2 == 0

    mesh = plsc.VectorSubcoreMesh(core_axis_name="c", subcore_axis_name="s")

    @functools.partial(
        pl.kernel,
        out_type=jax.ShapeDtypeStruct((HIST, n_batch, EMB), jnp.float32),
        mesh=mesh,
        scratch_types=[
            pltpu.VMEM((2, CH, NT, BB), jnp.int32),
            pltpu.VMEM((2, CH, NT, BB, EMB), jnp.float32),
            pltpu.VMEM((2, CH, BB, EMB), jnp.float32),
            pltpu.SemaphoreType.DMA,
            pltpu.SemaphoreType.DMA,
            pltpu.SemaphoreType.DMA,
            pltpu.SemaphoreType.DMA,
            pltpu.SemaphoreType.DMA,
            pltpu.SemaphoreType.DMA,
        ],
        compiler_params=pltpu.CompilerParams(
            use_tc_tiling_on_sc=False, needs_layout_passes=False),
    )
    def lookup(idx_hbm, w0, w1, w2, w3, out_hbm,
               idx_v, rows_v, out_v, gs0, gs1, is0, is1, os0, os1):
        tables = (w0, w1, w2, w3)
        gsem = (gs0, gs1)
        isem = (is0, is1)
        osem = (os0, os1)
        wid = lax.axis_index("s") * info.num_cores + lax.axis_index("c")
        b0 = wid * BB
        def idx_copies(k, sp):
            return [pltpu.make_async_copy(
                idx_hbm.at[pl.ds(k * CH, CH), :, pl.ds(b0, BB)],
                idx_v.at[sp], isem[sp])]

        def gather_copies(k, sp):
            del k
            return [pltpu.make_async_copy(
                tables[t].at[idx_v.at[sp, c, t]],
                rows_v.at[sp, c, t], gsem[sp])
                for c in range(CH) for t in range(NT)]

        def out_copies(k, sp):
            return [pltpu.make_async_copy(
                out_v.at[sp],
                out_hbm.at[pl.ds(k * CH, CH), pl.ds(b0, BB), :], osem[sp])]

        for c in idx_copies(0, 0):
            c.start()
        for c in idx_copies(0, 0):
            c.wait()
        for c in gather_copies(0, 0):
            c.start()
        for c in idx_copies(1, 1):
            c.start()

        def pair_body(kk, carry):
            for s in (0, 1):
                k = 2 * kk + s
                sn = 1 - s
                for c in gather_copies(k, s):
                    c.wait()

                @pl.when(k + 2 < n_chunks)
                def _prefetch_idx():
                    for c in idx_copies(k + 2, s):
                        c.start()

                @pl.when(k + 1 < n_chunks)
                def _fire_next():
                    for c in idx_copies(k + 1, sn):
                        c.wait()
                    for c in gather_copies(k + 1, sn):
                        c.start()

                @pl.when(k >= 2)
                def _reclaim_out():
                    for c in out_copies(k - 2, s):
                        c.wait()

                for c in range(CH):
                    def b_body(j, carry2, c=c):
                        for h in (0, EMB // 2):
                            d = pl.ds(h, EMB // 2)
                            s01 = (rows_v[s, c, 0, j, d]
                                   + rows_v[s, c, 1, j, d])
                            s23 = (rows_v[s, c, 2, j, d]
                                   + rows_v[s, c, 3, j, d])
                            out_v[s, c, j, d] = (
                                (s01 + s23) * jnp.float32(0.25))
                        return carry2

                    lax.fori_loop(0, BB, b_body, 0, unroll=8)
                for c in out_copies(k, s):
                    c.start()
            return carry

        lax.fori_loop(0, n_chunks // 2, pair_body, 0)
        for c in out_copies(n_chunks - 2, 0):
            c.wait()
        for c in out_copies(n_chunks - 1, 1):
            c.wait()

    return lookup


def kernel(poi_path, W0, W1, W2, W3):
    b, h, nt = poi_path.shape
    idx_t = jnp.transpose(poi_path, (1, 2, 0))
    out = _make_lookup(b)(idx_t, W0, W1, W2, W3)
    return jnp.transpose(out, (1, 0, 2))
